# Initial kernel scaffold; baseline (speedup 1.0000x reference)
import jax
import jax.numpy as jnp
from jax.experimental import pallas as pl


def kernel(E, A0):
    raise NotImplementedError("write your pallas kernel here")


# speedup vs baseline: 8.5397x; 8.5397x over previous
import functools
import math

import jax
import jax.numpy as jnp
import numpy as np
from jax import lax
from jax.experimental import pallas as pl
from jax.experimental.pallas import tpu as pltpu
from jax.experimental.pallas import tpu_sc as plsc

H = 32
W = 32
N = H * W
K = 32
KO = 32
TAU = 1.5
ETA = 8.0
LAM0 = 0.001
ALPHA = 10.0
BETA = 8.0
BETAP = 8.0
KS = 5
SIG = 1.0
EPS = 1e-06
B = 4
RBLK = 256
NRB = N // RBLK
INV2T = 1.0 / (2.0 * TAU ** 2 + 1e-12)


def _bins():
    maxr = (H ** 2 + W ** 2) ** 0.5 * 0.5
    ks = np.arange(1, KO + 1, dtype=np.float32)
    ang = 2.0 * np.pi * (ks - 1) / KO
    rad = 0.2 * maxr + 0.6 * maxr * (ks - 1) / (KO - 1 + 1e-09)
    return (rad * np.cos(ang)).astype(np.float32), (rad * np.sin(ang)).astype(np.float32)


_BX, _BY = _bins()


def _gauss():
    ax = np.arange(-KS // 2 + 1, KS // 2 + 1, dtype=np.float32)
    xx, yy = np.meshgrid(ax, ax, indexing="ij")
    k = np.exp(-(xx ** 2 + yy ** 2) / (2.0 * SIG ** 2))
    return (k / k.sum()).astype(np.float32)


_GW = _gauss()


def _passA_body(a_ref, vals_ref, idx_ref, hacc_ref, scratch):
    rb = pl.program_id(1)
    scratch[...] = a_ref[0]
    col = lax.broadcasted_iota(jnp.int32, (RBLK, N), 1)
    vals_l, idx_l = [], []
    for _ in range(K):
        v = scratch[...]
        mrow = jnp.max(v, axis=1, keepdims=True)
        im = jnp.where(v == mrow, col, N)
        ix = jnp.min(im, axis=1)
        vals_l.append(mrow[:, 0])
        idx_l.append(ix)
        scratch[...] = jnp.where(col == ix[:, None], -1.0, v)
    vals = jnp.stack(vals_l, axis=1)
    idx = jnp.stack(idx_l, axis=1)
    vals_ref[0] = vals
    idx_ref[0] = idx

    g = rb * RBLK + lax.broadcasted_iota(jnp.int32, (RBLK, K), 0)
    xi = (g % W).astype(jnp.float32)
    yi = (g // W).astype(jnp.float32)
    vx = (idx % W).astype(jnp.float32) - xi
    vy = (idx // W).astype(jnp.float32) - yi
    hs = []
    for ko in range(KO):
        d2 = (vx - _BX[ko]) ** 2 + (vy - _BY[ko]) ** 2
        kap = jnp.exp(-d2 * INV2T)
        hs.append(jnp.sum(vals * kap))
    hacc_ref[0, 0, 0] = jnp.stack(hs)


def _passA(A0):
    return pl.pallas_call(
        _passA_body,
        grid=(B, NRB),
        in_specs=[pl.BlockSpec((1, RBLK, N), lambda b, r: (b, r, 0))],
        out_specs=[
            pl.BlockSpec((1, RBLK, K), lambda b, r: (b, r, 0)),
            pl.BlockSpec((1, RBLK, K), lambda b, r: (b, r, 0)),
            pl.BlockSpec((1, 1, 1, KO), lambda b, r: (b, r, 0, 0)),
        ],
        out_shape=[
            jax.ShapeDtypeStruct((B, N, K), jnp.float32),
            jax.ShapeDtypeStruct((B, N, K), jnp.int32),
            jax.ShapeDtypeStruct((B, NRB, 1, KO), jnp.float32),
        ],
        scratch_shapes=[pltpu.VMEM((RBLK, N), jnp.float32)],
    )(A0)


def _passB_body(vals_ref, idx_ref, hacc_ref, a1_ref, chi_ref, u_ref, m_ref):
    b = pl.program_id(0)
    rb = pl.program_id(1)
    hacc = jnp.sum(hacc_ref[...], axis=(1, 2))
    z = ETA * hacc
    z = z - jnp.max(z, axis=1, keepdims=True)
    e = jnp.exp(z)
    wall = e / jnp.sum(e, axis=1, keepdims=True)
    wmean = jnp.mean(wall, axis=0)
    bsel = lax.broadcasted_iota(jnp.int32, (B, KO), 0) == b
    wsm = jnp.sum(jnp.where(bsel, wall, 0.0), axis=0)

    vals = vals_ref[0]
    idx = idx_ref[0]
    g = rb * RBLK + lax.broadcasted_iota(jnp.int32, (RBLK, K), 0)
    xi = (g % W).astype(jnp.float32)
    yi = (g // W).astype(jnp.float32)
    vx = (idx % W).astype(jnp.float32) - xi
    vy = (idx // W).astype(jnp.float32) - yi
    accR = jnp.zeros((RBLK, K), jnp.float32)
    accC = jnp.zeros((RBLK, K), jnp.float32)
    for ko in range(KO):
        d2 = (vx - _BX[ko]) ** 2 + (vy - _BY[ko]) ** 2
        kap = jnp.exp(-d2 * INV2T)
        accR = accR + wsm[ko] * kap
        accC = accC + wmean[ko] * kap
    tilde = vals * (LAM0 + accR)
    a1 = tilde / (jnp.sum(tilde, axis=1, keepdims=True) + EPS)
    zp = ALPHA * a1
    zp = zp - jnp.max(zp, axis=1, keepdims=True)
    ep = jnp.exp(zp)
    p = ep / jnp.sum(ep, axis=1, keepdims=True)
    ent = -jnp.sum(p * jnp.log(p + EPS), axis=1)
    u = 1.0 / (1.0 + jnp.exp(ent))
    a1_ref[0] = a1
    chi_ref[0] = accC
    u_ref[0] = u.reshape(RBLK // 32, 32)
    m_ref[0] = (1.0 - u).reshape(RBLK // 32, 32)


def _passB(vals, idx, hacc):
    return pl.pallas_call(
        _passB_body,
        grid=(B, NRB),
        in_specs=[
            pl.BlockSpec((1, RBLK, K), lambda b, r: (b, r, 0)),
            pl.BlockSpec((1, RBLK, K), lambda b, r: (b, r, 0)),
            pl.BlockSpec((B, NRB, 1, KO), lambda b, r: (0, 0, 0, 0)),
        ],
        out_specs=[
            pl.BlockSpec((1, RBLK, K), lambda b, r: (b, r, 0)),
            pl.BlockSpec((1, RBLK, K), lambda b, r: (b, r, 0)),
            pl.BlockSpec((1, RBLK // 32, 32), lambda b, r: (b, r, 0)),
            pl.BlockSpec((1, RBLK // 32, 32), lambda b, r: (b, r, 0)),
        ],
        out_shape=[
            jax.ShapeDtypeStruct((B, N, K), jnp.float32),
            jax.ShapeDtypeStruct((B, N, K), jnp.float32),
            jax.ShapeDtypeStruct((B, N // 32, 32), jnp.float32),
            jax.ShapeDtypeStruct((B, N // 32, 32), jnp.float32),
        ],
    )(vals, idx, hacc)


_NTILE = 32
_RPT = B * N // _NTILE
_EPT = _RPT * K
_GRP = 16
_NGRP = _RPT // _GRP
_CH = 128
_NCH = _EPT // _CH


def _passC_sc(idxf, a1f, chif, mf):
    mesh = plsc.VectorSubcoreMesh(core_axis_name="c", subcore_axis_name="s")
    out_type = [
        jax.ShapeDtypeStruct((B * N * N,), jnp.float32),
        jax.ShapeDtypeStruct((B * N,), jnp.float32),
        jax.ShapeDtypeStruct((B * N,), jnp.float32),
        jax.ShapeDtypeStruct((B * N,), jnp.float32),
    ]
    scratch = [
        pltpu.VMEM((_EPT,), jnp.int32),
        pltpu.VMEM((_EPT,), jnp.float32),
        pltpu.VMEM((_EPT,), jnp.float32),
        pltpu.VMEM((N,), jnp.float32),
        pltpu.VMEM((_EPT,), jnp.float32),
        pltpu.VMEM((_GRP * N,), jnp.float32),
        pltpu.VMEM((_RPT,), jnp.float32),
        pltpu.VMEM((_RPT,), jnp.float32),
        pltpu.VMEM((_NCH, _CH), jnp.int32),
        pltpu.VMEM((_EPT,), jnp.float32),
        pltpu.VMEM((_NCH, _CH), jnp.int32),
        pltpu.VMEM((_EPT,), jnp.float32),
        pltpu.VMEM((_RPT,), jnp.float32),
        pltpu.VMEM((_RPT,), jnp.float32),
        pltpu.VMEM((_RPT,), jnp.float32),
        pltpu.VMEM_SHARED((2 * N,), jnp.float32),
        pltpu.SemaphoreType.DMA,
    ]

    @functools.partial(pl.kernel, out_type=out_type, mesh=mesh,
                       scratch_types=scratch,
                       compiler_params=pltpu.CompilerParams(
                           needs_layout_passes=False))
    def body(idx_h, a1_h, chi_h, m_h, a2_h, q_h, pis_h, pit_h,
             idx_v, a1_v, chi_v, m_v, af_v, rowbuf, q_v, sev_v, gidx2_v,
             arev_v, tidx2_v, tcon_v, tve_v, pis_v, pit_v, t_sh, sem):
        cid = lax.axis_index("c")
        sid = lax.axis_index("s")
        grs = cid * (2 * N) + sid * _RPT
        bl = sid >> 3
        bg = cid * 2 + bl
        rwb0 = (sid & 7) * _RPT
        lane = lax.iota(jnp.int32, 16)
        mask0 = lane == 0

        pltpu.sync_copy(idx_h.at[pl.ds(grs * K, _EPT)], idx_v)
        pltpu.sync_copy(a1_h.at[pl.ds(grs * K, _EPT)], a1_v)
        pltpu.sync_copy(chi_h.at[pl.ds(grs * K, _EPT)], chi_v)
        pltpu.sync_copy(m_h.at[pl.ds(bg * N, N)], m_v)

        def z8(i, _):
            tve_v[pl.ds(i * 16, 16)] = jnp.zeros((16,), jnp.float32)
            return 0
        lax.fori_loop(0, _RPT // 16, z8, 0)
        pltpu.sync_copy(tve_v, t_sh.at[pl.ds(sid * _RPT, _RPT)])

        def group(g, _):
            def zrow(v, _):
                rowbuf[pl.ds(v * 16, 16)] = jnp.zeros((16,), jnp.float32)
                return 0
            lax.fori_loop(0, _GRP * N // 16, zrow, 0)

            def row(r, _):
                rr = g * _GRP + r
                base = rr * K
                i0 = idx_v[pl.ds(base, 16)]
                i1 = idx_v[pl.ds(base + 16, 16)]
                a10 = a1_v[pl.ds(base, 16)]
                a11 = a1_v[pl.ds(base + 16, 16)]
                mj0 = plsc.load_gather(m_v, [i0])
                mj1 = plsc.load_gather(m_v, [i1])
                rwb = rwb0 + rr
                mi = plsc.load_gather(m_v, [jnp.full((16,), rwb, jnp.int32)])
                h0 = mi * a10 * mj0
                h1 = mi * a11 * mj1
                s1 = jnp.sum(h0) + jnp.sum(h1) + EPS
                a2e0 = h0 / s1
                a2e1 = h1 / s1
                den = jnp.sum(a2e0) + jnp.sum(a2e1) + EPS
                af0 = a2e0 / den
                af1 = a2e1 / den
                af_v[pl.ds(base, 16)] = af0
                af_v[pl.ds(base + 16, 16)] = af1
                mx = jnp.maximum(jnp.max(a2e0), jnp.max(a2e1))
                e0 = jnp.exp(BETAP * (a2e0 - mx))
                e1 = jnp.exp(BETAP * (a2e1 - mx))
                se = jnp.sum(e0) + jnp.sum(e1)
                c0 = chi_v[pl.ds(base, 16)]
                c1 = chi_v[pl.ds(base + 16, 16)]
                qn = jnp.sum(e0 * c0) + jnp.sum(e1 * c1)
                qv = jnp.full((16,), qn, jnp.float32) / jnp.full((16,), se, jnp.float32)
                plsc.store_scatter(q_v, [jnp.full((16,), rr, jnp.int32)],
                                   qv, mask=mask0)
                roff = r * N
                plsc.store_scatter(rowbuf, [roff + i0], af0)
                plsc.store_scatter(rowbuf, [roff + i1], af1)
                return 0
            lax.fori_loop(0, _GRP, row, 0)
            pltpu.sync_copy(rowbuf,
                            a2_h.at[pl.ds((grs + g * _GRP) * N, _GRP * N)])
            return 0
        lax.fori_loop(0, _NGRP, group, 0)

        def bidx(e, _):
            iv = idx_v[pl.ds(e * 16, 16)]
            rwb = rwb0 + (e >> 1)
            gidx2_v[e >> 3, pl.ds((e & 7) * 16, 16)] = bg * (N * N) + iv * N + rwb
            tidx2_v[e >> 3, pl.ds((e & 7) * 16, 16)] = bl * N + iv
            return 0
        lax.fori_loop(0, _EPT // 16, bidx, 0)

        plsc.subcore_barrier()

        for w in range(_NCH // 8):
            cps = []
            for c8 in range(8):
                ch = w * 8 + c8
                cps.append(pltpu.async_copy(
                    a2_h.at[gidx2_v.at[ch]],
                    arev_v.at[pl.ds(ch * _CH, _CH)], sem))
            for cp in cps:
                cp.wait()

        def prow(rr, _):
            base = rr * K
            af0 = af_v[pl.ds(base, 16)]
            af1 = af_v[pl.ds(base + 16, 16)]
            ar0 = arev_v[pl.ds(base, 16)]
            ar1 = arev_v[pl.ds(base + 16, 16)]
            i0 = idx_v[pl.ds(base, 16)]
            i1 = idx_v[pl.ds(base + 16, 16)]
            rwb = rwb0 + rr
            ef0 = jnp.exp(BETA * af0)
            ef1 = jnp.exp(BETA * af1)
            er0 = jnp.exp(BETA * ar0)
            er1 = jnp.exp(BETA * ar1)
            P0 = ef0 / (ef0 + er0 + EPS)
            P1 = ef1 / (ef1 + er1 + EPS)
            P0 = jnp.where(i0 == rwb, 0.0, P0)
            P1 = jnp.where(i1 == rwb, 0.0, P1)
            sev = jnp.sum(af0 * P0) + jnp.sum(af1 * P1)
            plsc.store_scatter(sev_v, [jnp.full((16,), rr, jnp.int32)],
                               jnp.full((16,), sev, jnp.float32), mask=mask0)
            tcon_v[pl.ds(base, 16)] = af0 * (1.0 - P0)
            tcon_v[pl.ds(base + 16, 16)] = af1 * (1.0 - P1)
            return 0
        lax.fori_loop(0, _RPT, prow, 0)

        for ch in range(_NCH):
            pltpu.sync_copy(tcon_v.at[pl.ds(ch * _CH, _CH)],
                            t_sh.at[tidx2_v.at[ch]], add=True)
        plsc.subcore_barrier()

        pltpu.sync_copy(t_sh.at[pl.ds(sid * _RPT, _RPT)], tve_v)

        def pp(i, _):
            s = sev_v[pl.ds(i * 16, 16)]
            t = tve_v[pl.ds(i * 16, 16)]
            ps = s / (s + t + EPS)
            pis_v[pl.ds(i * 16, 16)] = ps
            pit_v[pl.ds(i * 16, 16)] = 1.0 - ps
            return 0
        lax.fori_loop(0, _RPT // 16, pp, 0)

        pltpu.sync_copy(q_v, q_h.at[pl.ds(grs, _RPT)])
        pltpu.sync_copy(pis_v, pis_h.at[pl.ds(grs, _RPT)])
        pltpu.sync_copy(pit_v, pit_h.at[pl.ds(grs, _RPT)])

    return body(idxf, a1f, chif, mf)


def _passC_jnp(idx, a1, chi, m):
    bi = jnp.arange(B)[:, None, None]
    ni = jnp.arange(N)[None, :, None]
    mj = jnp.take_along_axis(jnp.broadcast_to(m[:, None, :], (B, N, N)), idx, axis=2)
    hatA = m[:, :, None] * a1 * mj
    a2e = hatA / (hatA.sum(-1, keepdims=True) + EPS)
    den = a2e.sum(-1, keepdims=True) + EPS
    af = a2e / den
    A2 = jnp.zeros((B, N, N), jnp.float32).at[bi, ni, idx].set(af)
    arev = A2[bi, idx, ni]
    ef = jnp.exp(BETA * af)
    er = jnp.exp(BETA * arev)
    P = ef / (ef + er + EPS)
    P = jnp.where(idx == ni, 0.0, P)
    s_ev = (af * P).sum(-1)
    tcon = af * (1.0 - P)
    t_ev = jnp.zeros((B, N)).at[bi[..., 0], idx.reshape(B, -1)].add(tcon.reshape(B, -1))
    pi_src = s_ev / (s_ev + t_ev + EPS)
    zp = BETAP * a2e
    zp = zp - zp.max(-1, keepdims=True)
    ep = jnp.exp(zp)
    wl = ep / ep.sum(-1, keepdims=True)
    q = (wl * chi).sum(-1)
    return A2, pi_src, 1.0 - pi_src, q


def _passD_body(q_ref, out_ref, pad):
    pad[...] = jnp.zeros((B, H + 4, W + 4), jnp.float32)
    pad[:, 2:2 + H, 2:2 + W] = q_ref[...]
    acc = jnp.zeros((B, H, W), jnp.float32)
    for dy in range(KS):
        for dx in range(KS):
            acc = acc + _GW[dy, dx] * pad[:, dy:dy + H, dx:dx + W]
    out_ref[...] = 1.0 / (1.0 + jnp.exp(-acc))


def _passD(q):
    return pl.pallas_call(
        _passD_body,
        out_shape=jax.ShapeDtypeStruct((B, H, W), jnp.float32),
        scratch_shapes=[pltpu.VMEM((B, H + 4, W + 4), jnp.float32)],
    )(q)


def kernel(E, A0):
    del E
    vals, idx, hacc = _passA(A0)
    a1, chi, u3, m3 = _passB(vals, idx, hacc)
    U = u3.reshape(B, N)
    m = m3.reshape(B, N)
    a2f, q, pi_src_f, pi_tgt_f = _passC_sc(
        idx.reshape(-1), a1.reshape(-1), chi.reshape(-1), m.reshape(-1))
    A2 = a2f.reshape(B, N, N)
    pi_src = pi_src_f.reshape(B, N)
    pi_tgt = pi_tgt_f.reshape(B, N)
    A1_map = _passD(q.reshape(B, H, W)).reshape(B, 1, H, W)
    return (A2, U, pi_src, pi_tgt, A1_map, A2)
